# baseline (device time: 51716 ns/iter reference)
import jax
import jax.numpy as jnp
from jax import lax
from jax.experimental import pallas as pl
from jax.experimental.pallas import tpu as pltpu

N_DEV = 16
B, SQ, D = 1, 512, 1024
HQ, HKV, DH = 8, 2, 128
GROUP = HQ // HKV
SCALE = 0.08838834764831843
CHUNK = SQ // N_DEV


def kernel(x, Wq, Wo, K_ext, V_ext):
    skv = K_ext.shape[1]
    x2 = x.reshape(SQ, D)
    k2 = K_ext.reshape(skv, HKV * DH)
    v2 = V_ext.reshape(skv, HKV * DH)

    def body(x_ref, wq_ref, wo_ref, k_ref, v_ref, out_ref,
             o_snd, l_scr, rs_o, rs_l, y_div, y_snd, ag_ref,
             o_ssem, o_rsem, l_ssem, l_rsem, g_ssem, g_rsem):
        my = lax.axis_index("i")

        barrier = pltpu.get_barrier_semaphore()
        for k in range(1, N_DEV):
            pl.semaphore_signal(barrier, inc=1,
                                device_id=((my + k) % N_DEV,),
                                device_id_type=pl.DeviceIdType.MESH)
        pl.semaphore_wait(barrier, N_DEV - 1)

        q = jnp.dot(x_ref[...], wq_ref[...],
                    preferred_element_type=jnp.float32)
        for h in range(HQ):
            g = h // GROUP
            qh = q[:, h * DH:(h + 1) * DH]
            kh = k_ref[:, g * DH:(g + 1) * DH]
            vh = v_ref[:, g * DH:(g + 1) * DH]
            s = lax.dot_general(qh, kh, (((1,), (1,)), ((), ())),
                                preferred_element_type=jnp.float32) * SCALE
            p = jnp.exp(s)
            l_scr[:, h:h + 1] = jnp.sum(p, axis=1, keepdims=True)
            o_snd[:, h * DH:(h + 1) * DH] = jnp.dot(
                p, vh, preferred_element_type=jnp.float32).astype(jnp.bfloat16)

        rd_o = [None] * N_DEV
        rd_l = [None] * N_DEV
        for k in range(1, N_DEV):
            peer = (my + k) % N_DEV
            rd_o[k] = pltpu.make_async_remote_copy(
                src_ref=o_snd.at[pl.ds(peer * CHUNK, CHUNK), :],
                dst_ref=rs_o.at[k],
                send_sem=o_ssem.at[k], recv_sem=o_rsem.at[k],
                device_id=(peer,), device_id_type=pl.DeviceIdType.MESH)
            rd_l[k] = pltpu.make_async_remote_copy(
                src_ref=l_scr.at[pl.ds(peer * CHUNK, CHUNK), :],
                dst_ref=rs_l.at[k],
                send_sem=l_ssem.at[k], recv_sem=l_rsem.at[k],
                device_id=(peer,), device_id_type=pl.DeviceIdType.MESH)
            rd_o[k].start()
            rd_l[k].start()

        o_acc = o_snd[pl.ds(my * CHUNK, CHUNK), :].astype(jnp.float32)
        l_acc = l_scr[pl.ds(my * CHUNK, CHUNK), :]
        for k in range(1, N_DEV):
            rd_o[k].wait()
            o_acc = o_acc + rs_o[k].astype(jnp.float32)
            rd_l[k].wait()
            l_acc = l_acc + rs_l[k]

        for h in range(HQ):
            y_div[:, h * DH:(h + 1) * DH] = (
                o_acc[:, h * DH:(h + 1) * DH] / l_acc[:, h:h + 1])
        y = jnp.dot(y_div[...], wo_ref[...],
                    preferred_element_type=jnp.float32)
        out_ref[pl.ds(my * CHUNK, CHUNK), :] = y
        y_snd[...] = y.astype(jnp.bfloat16)

        rd_g = [None] * N_DEV
        for k in range(1, N_DEV):
            peer = (my + k) % N_DEV
            rd_g[k] = pltpu.make_async_remote_copy(
                src_ref=y_snd, dst_ref=ag_ref.at[k],
                send_sem=g_ssem.at[k], recv_sem=g_rsem.at[k],
                device_id=(peer,), device_id_type=pl.DeviceIdType.MESH)
            rd_g[k].start()
        for k in range(1, N_DEV):
            rd_g[k].wait()
            c = ((my - k) % N_DEV) * CHUNK
            out_ref[pl.ds(c, CHUNK), :] = ag_ref[k].astype(jnp.float32)

    out = pl.pallas_call(
        body,
        out_shape=jax.ShapeDtypeStruct((SQ, D), jnp.float32),
        in_specs=[pl.BlockSpec(memory_space=pltpu.VMEM)] * 5,
        out_specs=pl.BlockSpec(memory_space=pltpu.VMEM),
        scratch_shapes=[
            pltpu.VMEM((SQ, D), jnp.bfloat16),
            pltpu.VMEM((SQ, HQ), jnp.float32),
            pltpu.VMEM((N_DEV, CHUNK, D), jnp.bfloat16),
            pltpu.VMEM((N_DEV, CHUNK, HQ), jnp.float32),
            pltpu.VMEM((CHUNK, D), jnp.float32),
            pltpu.VMEM((CHUNK, D), jnp.bfloat16),
            pltpu.VMEM((N_DEV, CHUNK, D), jnp.bfloat16),
            pltpu.SemaphoreType.DMA((N_DEV,)),
            pltpu.SemaphoreType.DMA((N_DEV,)),
            pltpu.SemaphoreType.DMA((N_DEV,)),
            pltpu.SemaphoreType.DMA((N_DEV,)),
            pltpu.SemaphoreType.DMA((N_DEV,)),
            pltpu.SemaphoreType.DMA((N_DEV,)),
        ],
        compiler_params=pltpu.CompilerParams(collective_id=0),
    )(x2, Wq, Wo, k2, v2)
    return out.reshape(B, SQ, D)
